# baseline (speedup 1.0000x reference)
import jax
import jax.numpy as jnp
from jax import lax
from jax.experimental import pallas as pl
from jax.experimental.pallas import tpu as pltpu
from jax.experimental.pallas import tpu_sc as plsc

B = 16384
C = 1000
NC = 2
NS = 16
NW = NC * NS
BPW = B // NW
R = 32
NCH = BPW // R
NBUF = 3

_IN_BOUNDS = "promise_in_bounds"
_LN2 = 0.6931471805599453
_SQRT2 = 1.4142135623730951


def _rot(x, lanes, sh):
    return x.at[(lanes + sh) & 15].get(mode=_IN_BOUNDS)


def _allsum(x, lanes):
    for sh in (8, 4, 2, 1):
        x = x + _rot(x, lanes, sh)
    return x


def _ln(r):
    bits = lax.bitcast_convert_type(r, jnp.int32)
    e = ((bits >> 23) & 0xFF) - 127
    m = lax.bitcast_convert_type((bits & 0x007FFFFF) | 0x3F800000,
                                 jnp.float32)
    big = m > _SQRT2
    m = jnp.where(big, m * 0.5, m)
    e = jnp.where(big, e + 1, e)
    f = m - 1.0
    s = f / (2.0 + f)
    s2 = s * s
    p = 2.0 * s * (1.0 + s2 * (1.0 / 3.0 + s2 * (0.2 + s2 * (1.0 / 7.0))))
    return e.astype(jnp.float32) * _LN2 + p


def _sc_body(x_hbm, tgt_hbm, out_hbm,
             tgt_v, b0, b1, b2, out_v, psum_v, s0, s1, s2):
    bufs = [b0, b1, b2]
    sems = [s0, s1, s2]
    wid = lax.axis_index("s") * NC + lax.axis_index("c")
    base = wid * BPW
    lanes = lax.iota(jnp.int32, 16)
    zeros16 = jnp.zeros((16,), jnp.int32)

    pltpu.sync_copy(tgt_hbm.at[pl.ds(base, BPW)], tgt_v)

    def start(ch):
        p = ch % NBUF
        return pltpu.async_copy(x_hbm.at[pl.ds(base + ch * R, R)],
                                bufs[p], sems[p])

    handles = {}
    for ch in range(NBUF - 1):
        handles[ch] = start(ch)

    for ch in range(NCH):
        if ch + NBUF - 1 < NCH:
            handles[ch + NBUF - 1] = start(ch + NBUF - 1)
        handles[ch].wait()
        buf = bufs[ch % NBUF]
        for gg in range(R // 16):
            g = ch * (R // 16) + gg
            sl = pl.ds(g * 16, 16)
            rloc = gg * 16 + lanes
            t16 = tgt_v[sl]
            xt = plsc.load_gather(buf, [rloc, t16])
            x0 = plsc.load_gather(buf, [rloc, zeros16])
            out16 = 1.0 + jnp.exp(x0 - xt)
            out_v[sl] = out16
            zmask = t16 == 0
            nz = _allsum(zmask.astype(jnp.int32), lanes)[0]

            @pl.when(nz > 0)
            def _():
                def colbody(c, acc):
                    col = plsc.load_gather(buf, [rloc,
                                                 jnp.broadcast_to(c, (16,))])
                    return acc + jnp.exp(col - x0)
                acc = lax.fori_loop(0, C, colbody,
                                    jnp.zeros((16,), jnp.float32))
                out_v[sl] = jnp.where(zmask, acc, out16)

    acc_ln = jnp.zeros((16,), jnp.float32)
    for g in range(BPW // 16):
        acc_ln = acc_ln + _ln(out_v[pl.ds(g * 16, 16)])
    psum_v[...] = acc_ln
    pltpu.sync_copy(psum_v, out_hbm.at[wid])


@jax.jit
def _sc_loss_partials(x2d, tgt):
    mesh = plsc.VectorSubcoreMesh(core_axis_name="c", subcore_axis_name="s",
                                  num_cores=NC, num_subcores=NS)
    return pl.kernel(
        _sc_body,
        out_type=jax.ShapeDtypeStruct((NW, 16), jnp.float32),
        mesh=mesh,
        scratch_types=[
            pltpu.VMEM((BPW,), jnp.int32),
            pltpu.VMEM((R, C), jnp.float32),
            pltpu.VMEM((R, C), jnp.float32),
            pltpu.VMEM((R, C), jnp.float32),
            pltpu.VMEM((BPW,), jnp.float32),
            pltpu.VMEM((16,), jnp.float32),
            pltpu.SemaphoreType.DMA,
            pltpu.SemaphoreType.DMA,
            pltpu.SemaphoreType.DMA,
        ],
        compiler_params=pltpu.CompilerParams(
            needs_layout_passes=False,
            use_tc_tiling_on_sc=True,
        ),
    )(x2d, tgt)


def kernel(inputs, targets):
    tgt = targets.astype(jnp.int32)
    partials = _sc_loss_partials(inputs, tgt)
    return jnp.sum(partials) * (1.0 / B)


# speedup vs baseline: 1.0452x; 1.0452x over previous
import jax
import jax.numpy as jnp
from jax import lax
from jax.experimental import pallas as pl
from jax.experimental.pallas import tpu as pltpu
from jax.experimental.pallas import tpu_sc as plsc

B = 16384
C = 1000
NC = 2
NS = 16
NW = NC * NS
BPW = B // NW
R = 32
NCH = BPW // R
NBUF = 3

_IN_BOUNDS = "promise_in_bounds"
_LN2 = 0.6931471805599453
_SQRT2 = 1.4142135623730951


def _rot(x, lanes, sh):
    return x.at[(lanes + sh) & 15].get(mode=_IN_BOUNDS)


def _allsum(x, lanes):
    for sh in (8, 4, 2, 1):
        x = x + _rot(x, lanes, sh)
    return x


def _ln(r):
    bits = lax.bitcast_convert_type(r, jnp.int32)
    e = ((bits >> 23) & 0xFF) - 127
    m = lax.bitcast_convert_type((bits & 0x007FFFFF) | 0x3F800000,
                                 jnp.float32)
    big = m > _SQRT2
    m = jnp.where(big, m * 0.5, m)
    e = jnp.where(big, e + 1, e)
    f = m - 1.0
    s = f / (2.0 + f)
    s2 = s * s
    p = 2.0 * s * (1.0 + s2 * (1.0 / 3.0 + s2 * (0.2 + s2 * (1.0 / 7.0))))
    return e.astype(jnp.float32) * _LN2 + p


def _sc_body(x_hbm, tgt_hbm, out_hbm,
             tgt_v, b0, b1, b2, out_v, psum_v, s0, s1, s2):
    bufs = [b0, b1, b2]
    sems = [s0, s1, s2]
    wid = lax.axis_index("s") * NC + lax.axis_index("c")
    base = wid * BPW
    lanes = lax.iota(jnp.int32, 16)
    zeros16 = jnp.zeros((16,), jnp.int32)

    pltpu.sync_copy(tgt_hbm.at[pl.ds(base, BPW)], tgt_v)

    def start(ch):
        p = ch % NBUF
        return pltpu.async_copy(
            x_hbm.at[pl.ds((base + ch * R) // 8, R // 8)],
            bufs[p], sems[p])

    handles = {}
    for ch in range(NBUF - 1):
        handles[ch] = start(ch)

    for ch in range(NCH):
        if ch + NBUF - 1 < NCH:
            handles[ch + NBUF - 1] = start(ch + NBUF - 1)
        handles[ch].wait()
        buf = bufs[ch % NBUF]
        for gg in range(R // 16):
            g = ch * (R // 16) + gg
            sl = pl.ds(g * 16, 16)
            rloc = gg * 16 + lanes
            t16 = tgt_v[sl]
            xt = plsc.load_gather(buf, [rloc >> 3, rloc & 7, t16])
            x0 = plsc.load_gather(buf, [rloc >> 3, rloc & 7, zeros16])
            out16 = 1.0 + jnp.exp(x0 - xt)
            out_v[sl] = out16
            zmask = t16 == 0
            nz = _allsum(zmask.astype(jnp.int32), lanes)[0]

            @pl.when(nz > 0)
            def _():
                def colbody(c, acc):
                    col = plsc.load_gather(buf, [rloc >> 3, rloc & 7,
                                                 jnp.broadcast_to(c, (16,))])
                    return acc + jnp.exp(col - x0)
                acc = lax.fori_loop(0, C, colbody,
                                    jnp.zeros((16,), jnp.float32))
                out_v[sl] = jnp.where(zmask, acc, out16)

    acc_ln = jnp.zeros((16,), jnp.float32)
    for g in range(BPW // 16):
        acc_ln = acc_ln + _ln(out_v[pl.ds(g * 16, 16)])
    psum_v[...] = acc_ln
    pltpu.sync_copy(psum_v, out_hbm.at[wid])


@jax.jit
def _sc_loss_partials(x2d, tgt):
    mesh = plsc.VectorSubcoreMesh(core_axis_name="c", subcore_axis_name="s",
                                  num_cores=NC, num_subcores=NS)
    return pl.kernel(
        _sc_body,
        out_type=jax.ShapeDtypeStruct((NW, 16), jnp.float32),
        mesh=mesh,
        scratch_types=[
            pltpu.VMEM((BPW,), jnp.int32),
            pltpu.VMEM((R // 8, 8, C), jnp.float32),
            pltpu.VMEM((R // 8, 8, C), jnp.float32),
            pltpu.VMEM((R // 8, 8, C), jnp.float32),
            pltpu.VMEM((BPW,), jnp.float32),
            pltpu.VMEM((16,), jnp.float32),
            pltpu.SemaphoreType.DMA,
            pltpu.SemaphoreType.DMA,
            pltpu.SemaphoreType.DMA,
        ],
        compiler_params=pltpu.CompilerParams(
            needs_layout_passes=False,
            use_tc_tiling_on_sc=True,
        ),
    )(x2d, tgt)


def kernel(inputs, targets):
    tgt = targets.astype(jnp.int32)
    partials = _sc_loss_partials(inputs.reshape(B // 8, 8, C), tgt)
    return jnp.sum(partials) * (1.0 / B)
